# baseline (speedup 1.0000x reference)
import jax
import jax.numpy as jnp
from jax import lax
from jax.experimental import pallas as pl
from jax.experimental.pallas import tpu as pltpu
from jax.experimental.pallas import tpu_sc as plsc

_NC, _NS = 2, 16
_NW = _NC * _NS
_CH = 8
_LANES = 16
_NBUF = 4
_LOOK = 2


def _sc_body(S, D, B, x_hbm, pe_hbm, out_hbm, xbufs, pbufs,
             lsems, ssems, psems):
    s_per_w = S // _NW
    n_ch = s_per_w // _CH
    w = lax.axis_index("s") * _NC + lax.axis_index("c")
    s0 = w * s_per_w

    load_d = [None] * _NBUF
    store_d = [None] * _NBUF

    def start_load(t):
        buf = t % _NBUF
        if store_d[buf] is not None:
            store_d[buf].wait()
            store_d[buf] = None
        load_d[buf] = pltpu.async_copy(
            x_hbm.at[:, pl.ds(s0 + t * _CH, _CH)], xbufs[buf],
            lsems.at[buf])

    def start_pe(t):
        return pltpu.async_copy(
            pe_hbm.at[pl.ds(s0 + t * _CH, _CH)],
            pbufs[t % 2], psems.at[t % 2])

    pe_d = [start_pe(0), start_pe(1)]
    for t0 in range(_LOOK):
        start_load(t0)

    for t in range(n_ch):
        buf = t % _NBUF
        pe_d[t % 2].wait()
        load_d[buf].wait()
        xbuf = xbufs[buf]
        pbuf = pbufs[t % 2]

        for b in range(B):
            @plsc.parallel_loop(0, _CH, 1)
            def _(i):
                @plsc.parallel_loop(0, D, _LANES, unroll=8)
                def _(k):
                    plsc.addupdate(xbuf.at[b, i, pl.ds(k, _LANES)],
                                   pbuf[i, pl.ds(k, _LANES)])

        if t + 2 < n_ch:
            pe_d[t % 2] = start_pe(t + 2)
        store_d[buf] = pltpu.async_copy(
            xbuf, out_hbm.at[:, pl.ds(s0 + t * _CH, _CH)], ssems.at[buf])
        if t + _LOOK < n_ch:
            start_load(t + _LOOK)

    for d in store_d:
        if d is not None:
            d.wait()


def kernel(x, pe):
    B, S, D = x.shape
    mesh = plsc.VectorSubcoreMesh(core_axis_name="c", subcore_axis_name="s")

    def body(x_hbm, pe_hbm, out_hbm, *scratch):
        _sc_body(S, D, B, x_hbm, pe_hbm, out_hbm,
                 scratch[:_NBUF], scratch[_NBUF:_NBUF + 2],
                 scratch[-3], scratch[-2], scratch[-1])

    run = pl.kernel(
        body,
        out_type=jax.ShapeDtypeStruct((B, S, D), x.dtype),
        mesh=mesh,
        scratch_types=(
            [pltpu.VMEM((B, _CH, D), jnp.float32) for _ in range(_NBUF)]
            + [pltpu.VMEM((_CH, D), jnp.float32) for _ in range(2)]
            + [pltpu.SemaphoreType.DMA((_NBUF,)),
               pltpu.SemaphoreType.DMA((_NBUF,)),
               pltpu.SemaphoreType.DMA((2,))]
        ),
    )
    return run(x, pe)


# speedup vs baseline: 1.0386x; 1.0386x over previous
import jax
import jax.numpy as jnp
from jax import lax
from jax.experimental import pallas as pl
from jax.experimental.pallas import tpu as pltpu
from jax.experimental.pallas import tpu_sc as plsc

_NC, _NS = 2, 16
_NW = _NC * _NS
_CH = 16
_LANES = 16
_NBUF = 8
_LOOK = 6


def _sc_body(S, D, B, x_hbm, pe_hbm, out_hbm, xbufs, pbufs,
             lsems, ssems, psems):
    s_per_w = S // _NW
    n_ch = s_per_w // _CH
    n_steps = n_ch * B
    w = lax.axis_index("s") * _NC + lax.axis_index("c")
    s0 = w * s_per_w

    load_d = [None] * _NBUF
    store_d = [None] * _NBUF

    def start_load(t):
        ic, b = divmod(t, B)
        buf = t % _NBUF
        if store_d[buf] is not None:
            store_d[buf].wait()
            store_d[buf] = None
        load_d[buf] = pltpu.async_copy(
            x_hbm.at[b, pl.ds(s0 + ic * _CH, _CH)], xbufs[buf],
            lsems.at[buf])

    def start_pe(ic):
        return pltpu.async_copy(
            pe_hbm.at[pl.ds(s0 + ic * _CH, _CH)],
            pbufs[ic % 2], psems.at[ic % 2])

    pe_d = [start_pe(0), None]
    for t0 in range(_LOOK):
        start_load(t0)

    for t in range(n_steps):
        ic, b = divmod(t, B)
        buf = t % _NBUF
        if b == 0:
            pe_d[ic % 2].wait()
            if ic + 1 < n_ch:
                pe_d[(ic + 1) % 2] = start_pe(ic + 1)
        load_d[buf].wait()
        xbuf = xbufs[buf]
        pbuf = pbufs[ic % 2]

        @plsc.parallel_loop(0, _CH, 1)
        def _(i):
            @plsc.parallel_loop(0, D, _LANES, unroll=8)
            def _(k):
                plsc.addupdate(xbuf.at[i, pl.ds(k, _LANES)],
                               pbuf[i, pl.ds(k, _LANES)])

        store_d[buf] = pltpu.async_copy(
            xbuf, out_hbm.at[b, pl.ds(s0 + ic * _CH, _CH)], ssems.at[buf])
        if t + _LOOK < n_steps:
            start_load(t + _LOOK)

    for d in store_d:
        if d is not None:
            d.wait()


def kernel(x, pe):
    B, S, D = x.shape
    mesh = plsc.VectorSubcoreMesh(core_axis_name="c", subcore_axis_name="s")

    def body(x_hbm, pe_hbm, out_hbm, *scratch):
        _sc_body(S, D, B, x_hbm, pe_hbm, out_hbm,
                 scratch[:_NBUF], scratch[_NBUF:_NBUF + 2],
                 scratch[-3], scratch[-2], scratch[-1])

    run = pl.kernel(
        body,
        out_type=jax.ShapeDtypeStruct((B, S, D), x.dtype),
        mesh=mesh,
        scratch_types=(
            [pltpu.VMEM((_CH, D), jnp.float32) for _ in range(_NBUF)]
            + [pltpu.VMEM((_CH, D), jnp.float32) for _ in range(2)]
            + [pltpu.SemaphoreType.DMA((_NBUF,)),
               pltpu.SemaphoreType.DMA((_NBUF,)),
               pltpu.SemaphoreType.DMA((2,))]
        ),
    )
    return run(x, pe)
